# baseline (speedup 1.0000x reference)
import jax
import jax.numpy as jnp
from jax.experimental import pallas as pl
from jax.experimental.pallas import tpu as pltpu

NSUB = 2
KCHUNKS = 4


def _body(x_ref, W1_ref, b1_ref, W2_ref, b2_ref, E_ref,
          Wd1_ref, bd1_ref, Wd2_ref, bd2_ref, out_ref):
    TB = x_ref.shape[0]
    L, K = E_ref.shape
    E = E_ref[...]
    h = jnp.maximum(
        jnp.dot(x_ref[...], W1_ref[...], preferred_element_type=jnp.float32)
        + b1_ref[...], 0.0)
    z = jnp.maximum(
        jnp.dot(h, W2_ref[...], preferred_element_type=jnp.float32)
        + b2_ref[...], 0.0)
    z_sq = jnp.sum(z * z, axis=1, keepdims=True)
    e_sq = jnp.sum(E * E, axis=0, keepdims=True)
    KW = K // KCHUNKS
    m = None
    idx = None
    for c in range(KCHUNKS):
        cols = slice(c * KW, (c + 1) * KW)
        sim_c = jnp.dot(z, E[:, cols], preferred_element_type=jnp.float32)
        dist_c = z_sq + e_sq[:, cols] - 2.0 * sim_c
        idx_c = jnp.argmin(dist_c, axis=1) + c * KW
        min_c = jnp.min(dist_c, axis=1)
        if m is None:
            m, idx = min_c, idx_c
        else:
            better = min_c < m
            idx = jnp.where(better, idx_c, idx)
            m = jnp.minimum(m, min_c)
    S = TB // NSUB
    for s in range(NSUB):
        r = slice(s * S, (s + 1) * S)
        k_iota = jax.lax.broadcasted_iota(jnp.int32, (S, K), 1)
        onehot = (k_iota == idx[r][:, None]).astype(jnp.float32)
        quant = jax.lax.dot_general(
            onehot, E, (((1,), (1,)), ((), ())),
            preferred_element_type=jnp.float32)
        zc = z[r]
        q = zc + (quant - zc)
        hd = jnp.maximum(
            jnp.dot(q, Wd1_ref[...], preferred_element_type=jnp.float32)
            + bd1_ref[...], 0.0)
        out_ref[pl.ds(s * S, S), :] = (
            jnp.dot(hd, Wd2_ref[...], preferred_element_type=jnp.float32)
            + bd2_ref[...])


@jax.jit
def kernel(x, W1, b1, W2, b2, E, Wd1, bd1, Wd2, bd2):
    B, D = x.shape
    L, K = E.shape
    Dh = W1.shape[1]
    TB = min(2048, B)
    grid = (B // TB,)

    def batch_map(i):
        return (i, 0)

    def const_map(i):
        return (0, 0)

    full = lambda shape: pl.BlockSpec(shape, const_map)
    out = pl.pallas_call(
        _body,
        grid=grid,
        in_specs=[
            pl.BlockSpec((TB, D), batch_map),
            full((D, Dh)),
            full((1, Dh)),
            full((Dh, L)),
            full((1, L)),
            full((L, K)),
            full((L, Dh)),
            full((1, Dh)),
            full((Dh, D)),
            full((1, D)),
        ],
        out_specs=pl.BlockSpec((TB, D), batch_map),
        out_shape=jax.ShapeDtypeStruct((B, D), jnp.float32),
        compiler_params=pltpu.CompilerParams(
            dimension_semantics=("parallel",),
        ),
    )(x, W1, b1.reshape(1, -1), W2, b2.reshape(1, -1), E,
      Wd1, bd1.reshape(1, -1), Wd2, bd2.reshape(1, -1))
    return out


# speedup vs baseline: 1.7958x; 1.7958x over previous
import jax
import jax.numpy as jnp
from jax.experimental import pallas as pl
from jax.experimental.pallas import tpu as pltpu

NSUB = 2


def _body(x_ref, W1_ref, b1_ref, W2_ref, b2_ref, E_ref,
          Wd1_ref, bd1_ref, Wd2_ref, bd2_ref, out_ref):
    TB = x_ref.shape[0]
    E = E_ref[...]
    h = jnp.maximum(
        jnp.dot(x_ref[...], W1_ref[...], preferred_element_type=jnp.float32)
        + b1_ref[...], 0.0)
    z = jnp.maximum(
        jnp.dot(h, W2_ref[...], preferred_element_type=jnp.float32)
        + b2_ref[...], 0.0)
    sim = jnp.dot(z, E, preferred_element_type=jnp.float32)
    z_sq = jnp.sum(z * z, axis=1, keepdims=True)
    e_sq = jnp.sum(E * E, axis=0, keepdims=True)
    S = TB // NSUB
    for s in range(NSUB):
        r = slice(s * S, (s + 1) * S)
        dist = z_sq[r] + e_sq - 2.0 * sim[r]
        idx = jnp.argmin(dist, axis=1)
        k_iota = jax.lax.broadcasted_iota(jnp.int32, dist.shape, 1)
        onehot = (k_iota == idx[:, None]).astype(jnp.float32)
        quant = jax.lax.dot_general(
            onehot, E, (((1,), (1,)), ((), ())),
            preferred_element_type=jnp.float32)
        zc = z[r]
        q = zc + (quant - zc)
        hd = jnp.maximum(
            jnp.dot(q, Wd1_ref[...], preferred_element_type=jnp.float32)
            + bd1_ref[...], 0.0)
        out_ref[pl.ds(s * S, S), :] = (
            jnp.dot(hd, Wd2_ref[...], preferred_element_type=jnp.float32)
            + bd2_ref[...])


@jax.jit
def kernel(x, W1, b1, W2, b2, E, Wd1, bd1, Wd2, bd2):
    B, D = x.shape
    L, K = E.shape
    Dh = W1.shape[1]
    TB = min(2048, B)
    grid = (B // TB,)

    def batch_map(i):
        return (i, 0)

    def const_map(i):
        return (0, 0)

    full = lambda shape: pl.BlockSpec(shape, const_map)
    out = pl.pallas_call(
        _body,
        grid=grid,
        in_specs=[
            pl.BlockSpec((TB, D), batch_map),
            full((D, Dh)),
            full((1, Dh)),
            full((Dh, L)),
            full((1, L)),
            full((L, K)),
            full((L, Dh)),
            full((1, Dh)),
            full((Dh, D)),
            full((1, D)),
        ],
        out_specs=pl.BlockSpec((TB, D), batch_map),
        out_shape=jax.ShapeDtypeStruct((B, D), jnp.float32),
        compiler_params=pltpu.CompilerParams(
            dimension_semantics=("parallel",),
        ),
    )(x, W1, b1.reshape(1, -1), W2, b2.reshape(1, -1), E,
      Wd1, bd1.reshape(1, -1), Wd2, bd2.reshape(1, -1))
    return out
